# baseline (speedup 1.0000x reference)
import functools

import jax
import jax.numpy as jnp
import numpy as np
from jax import lax
from jax.experimental import pallas as pl
from jax.experimental.pallas import tpu as pltpu
from jax.experimental.pallas import tpu_sc as plsc

_HIDDEN = 128
_ENT_DIM = 256
_B = 1024
_NNEG = 256
_GAMMA = 12.0
_EMB_RANGE = (12.0 + 2.0) / _HIDDEN
_PHASE_K = float(np.pi) / _EMB_RANGE

_TILES = 32
_B_PER_TILE = _B // _TILES
_ROWS_PER_TILE = _B_PER_TILE * _NNEG
_CHUNK = 128
_PAIRS = _ROWS_PER_TILE // (2 * _CHUNK)
_NROWS = _B * _NNEG

_SIN_C = (9.9999999443e-01, -1.6666664567e-01, 8.3333102843e-03,
          -1.9840151690e-04, 2.7529392628e-06, -2.4676469125e-08,
          1.3449911084e-10)
_COS_C = (1.0000000001e+00, -4.9999999854e-01, 4.1666663479e-02,
          -1.3888863033e-03, 2.4800553772e-05, -2.7534807478e-07,
          2.0603622903e-09, -9.7225822060e-12)

_MAGIC = np.int32(0x5F3759DF)


def _sin_poly(t):
    t2 = t * t
    r = jnp.float32(_SIN_C[-1])
    for c in _SIN_C[-2::-1]:
        r = r * t2 + jnp.float32(c)
    return r * t


def _cos_poly(t):
    t2 = t * t
    r = jnp.float32(_COS_C[-1])
    for c in _COS_C[-2::-1]:
        r = r * t2 + jnp.float32(c)
    return r


def _sqrt_fast(s):
    i = lax.bitcast_convert_type(s, jnp.int32)
    y = lax.bitcast_convert_type(_MAGIC - (i >> 1), jnp.float32)
    y = y * (1.5 - 0.5 * s * y * y)
    y = y * (1.5 - 0.5 * s * y * y)
    return s * y


def _sc_body(head_idx_h, rel_idx_h, tail_idx_h, ent_h, relemb_h, out_h,
             hidx_v, ridx_v, idx_v, head_v, rel_v, rot_re_v, rot_im_v,
             buf0, buf1, part_v, sem0, sem_a, sem_b):
    wid = lax.axis_index("s") * 2 + lax.axis_index("c")
    tb = wid * _B_PER_TILE

    pltpu.sync_copy(head_idx_h.at[pl.ds(tb, _B_PER_TILE)], hidx_v)
    pltpu.sync_copy(rel_idx_h.at[pl.ds(tb, _B_PER_TILE)], ridx_v)
    pltpu.sync_copy(tail_idx_h.at[pl.ds(tb * _NNEG, _ROWS_PER_TILE)], idx_v)
    pltpu.async_copy(ent_h.at[hidx_v], head_v, sem0).wait()
    pltpu.async_copy(relemb_h.at[ridx_v], rel_v, sem0).wait()

    def rot_body(b, carry):
        for hv in range(_HIDDEN // 16):
            sl = pl.ds(hv * 16, 16)
            ph = rel_v[b, sl] * jnp.float32(_PHASE_K)
            cr = _cos_poly(ph)
            sr = _sin_poly(ph)
            rh = head_v[b, sl]
            ih = head_v[b, pl.ds(_HIDDEN + hv * 16, 16)]
            dst = pl.ds(b * _HIDDEN + hv * 16, 16)
            rot_re_v[dst] = rh * cr - ih * sr
            rot_im_v[dst] = rh * sr + ih * cr
        return carry

    lax.fori_loop(0, _B_PER_TILE, rot_body, 0)

    def start_chunk(c, buf, sem):
        pltpu.make_async_copy(
            ent_h.at[idx_v.at[pl.ds(pl.multiple_of(c * _CHUNK, _CHUNK),
                                    _CHUNK)]],
            buf, sem).start()

    def wait_chunk(buf, sem):
        pltpu.make_async_copy(
            ent_h.at[idx_v.at[pl.ds(0, _CHUNK)]], buf, sem).wait()

    def compute_chunk(buf, b, c):
        rotr = [rot_re_v[pl.ds(b * _HIDDEN + hv * 16, 16)]
                for hv in range(_HIDDEN // 16)]
        roti = [rot_im_v[pl.ds(b * _HIDDEN + hv * 16, 16)]
                for hv in range(_HIDDEN // 16)]

        def row_body(j, carry):
            sq = []
            for hv in range(_HIDDEN // 16):
                rt = buf[j, pl.ds(hv * 16, 16)]
                it = buf[j, pl.ds(_HIDDEN + hv * 16, 16)]
                d1 = rotr[hv] - rt
                d2 = roti[hv] - it
                s = d1 * d1 + (d2 * d2 + 1e-35)
                sq.append(_sqrt_fast(s))
            part = (((sq[0] + sq[1]) + (sq[2] + sq[3]))
                    + ((sq[4] + sq[5]) + (sq[6] + sq[7])))
            part_v[pl.ds(j * 16, 16)] = part
            return carry

        lax.fori_loop(0, _CHUNK, row_body, 0)
        pltpu.sync_copy(
            part_v,
            out_h.at[pl.ds((tb * _NNEG + c * _CHUNK) * 16, _CHUNK * 16)])

    start_chunk(0, buf0, sem_a)

    def pair_body(i, carry):
        start_chunk(2 * i + 1, buf1, sem_b)
        wait_chunk(buf0, sem_a)
        compute_chunk(buf0, i, 2 * i)

        @pl.when(i < _PAIRS - 1)
        def _():
            start_chunk(2 * i + 2, buf0, sem_a)

        wait_chunk(buf1, sem_b)
        compute_chunk(buf1, i, 2 * i + 1)
        return carry

    lax.fori_loop(0, _PAIRS, pair_body, 0)


@functools.lru_cache(maxsize=1)
def _build_scoring():
    return functools.partial(
        pl.kernel,
        out_type=jax.ShapeDtypeStruct((_NROWS * 16,), jnp.float32),
        scratch_types=[
            pltpu.VMEM((_B_PER_TILE,), jnp.int32),
            pltpu.VMEM((_B_PER_TILE,), jnp.int32),
            pltpu.VMEM((_ROWS_PER_TILE,), jnp.int32),
            pltpu.VMEM((_B_PER_TILE, _ENT_DIM), jnp.float32),
            pltpu.VMEM((_B_PER_TILE, _HIDDEN), jnp.float32),
            pltpu.VMEM((_B_PER_TILE * _HIDDEN,), jnp.float32),
            pltpu.VMEM((_B_PER_TILE * _HIDDEN,), jnp.float32),
            pltpu.VMEM((_CHUNK, _ENT_DIM), jnp.float32),
            pltpu.VMEM((_CHUNK, _ENT_DIM), jnp.float32),
            pltpu.VMEM((_CHUNK * 16,), jnp.float32),
            pltpu.SemaphoreType.DMA,
            pltpu.SemaphoreType.DMA,
            pltpu.SemaphoreType.DMA,
        ],
        mesh=plsc.VectorSubcoreMesh(core_axis_name="c", subcore_axis_name="s"),
    )(_sc_body)


_RED_ROWS = 8192


def _tc_reduce_body(part_ref, out_ref):
    x = part_ref[...]
    s = jnp.float32(_GAMMA) - jnp.sum(x, axis=1)
    out_ref[...] = s.reshape(8, _RED_ROWS // 8)


@functools.lru_cache(maxsize=1)
def _build_reduce():
    grid = _NROWS // _RED_ROWS
    return pl.pallas_call(
        _tc_reduce_body,
        grid=(grid,),
        in_specs=[pl.BlockSpec((_RED_ROWS, 16), lambda i: (i, 0))],
        out_specs=pl.BlockSpec((8, _RED_ROWS // 8), lambda i: (i, 0)),
        out_shape=jax.ShapeDtypeStruct((grid * 8, _RED_ROWS // 8),
                                       jnp.float32),
    )


@jax.jit
def kernel(head_idx, rel_idx, neg_tail_idx, entity_embedding,
           relation_embedding):
    tail_flat = neg_tail_idx.reshape(-1)
    part = _build_scoring()(head_idx, rel_idx, tail_flat, entity_embedding,
                            relation_embedding)
    out = _build_reduce()(part.reshape(_NROWS, 16))
    return out.reshape(_B, _NNEG)


# speedup vs baseline: 1.1035x; 1.1035x over previous
import functools

import jax
import jax.numpy as jnp
import numpy as np
from jax import lax
from jax.experimental import pallas as pl
from jax.experimental.pallas import tpu as pltpu
from jax.experimental.pallas import tpu_sc as plsc

_HIDDEN = 128
_ENT_DIM = 256
_B = 1024
_NNEG = 256
_GAMMA = 12.0
_EMB_RANGE = (12.0 + 2.0) / _HIDDEN
_PHASE_K = float(np.pi) / _EMB_RANGE

_TILES = 32
_B_PER_TILE = _B // _TILES
_ROWS_PER_TILE = _B_PER_TILE * _NNEG
_CHUNK = 128
_PAIRS = _ROWS_PER_TILE // (2 * _CHUNK)
_NROWS = _B * _NNEG

_SIN_C = (9.9999999443e-01, -1.6666664567e-01, 8.3333102843e-03,
          -1.9840151690e-04, 2.7529392628e-06, -2.4676469125e-08,
          1.3449911084e-10)
_COS_C = (1.0000000001e+00, -4.9999999854e-01, 4.1666663479e-02,
          -1.3888863033e-03, 2.4800553772e-05, -2.7534807478e-07,
          2.0603622903e-09, -9.7225822060e-12)

_MAGIC = np.int32(0x5F3759DF)


def _sin_poly(t):
    t2 = t * t
    r = jnp.float32(_SIN_C[-1])
    for c in _SIN_C[-2::-1]:
        r = r * t2 + jnp.float32(c)
    return r * t


def _cos_poly(t):
    t2 = t * t
    r = jnp.float32(_COS_C[-1])
    for c in _COS_C[-2::-1]:
        r = r * t2 + jnp.float32(c)
    return r


def _sqrt_fast(s):
    i = lax.bitcast_convert_type(s, jnp.int32)
    y = lax.bitcast_convert_type(_MAGIC - (i >> 1), jnp.float32)
    y = y * (1.5 - 0.5 * s * y * y)
    y = y * (1.5 - 0.5 * s * y * y)
    return s * y


def _sc_body(head_idx_h, rel_idx_h, tail_idx_h, ent_h, relemb_h, out_h,
             hidx_v, ridx_v, idx_v, head_v, rel_v, rot_re_v, rot_im_v,
             buf0, buf1, part_v, sem0, sem_a, sem_b):
    wid = lax.axis_index("s") * 2 + lax.axis_index("c")
    tb = wid * _B_PER_TILE

    pltpu.sync_copy(head_idx_h.at[pl.ds(tb, _B_PER_TILE)], hidx_v)
    pltpu.sync_copy(rel_idx_h.at[pl.ds(tb, _B_PER_TILE)], ridx_v)
    pltpu.sync_copy(tail_idx_h.at[pl.ds(tb * _NNEG, _ROWS_PER_TILE)], idx_v)
    pltpu.async_copy(ent_h.at[hidx_v], head_v, sem0).wait()
    pltpu.async_copy(relemb_h.at[ridx_v], rel_v, sem0).wait()

    def rot_body(b, carry):
        for hv in range(_HIDDEN // 16):
            sl = pl.ds(hv * 16, 16)
            ph = rel_v[b, sl] * jnp.float32(_PHASE_K)
            cr = _cos_poly(ph)
            sr = _sin_poly(ph)
            rh = head_v[b, sl]
            ih = head_v[b, pl.ds(_HIDDEN + hv * 16, 16)]
            dst = pl.ds(b * _HIDDEN + hv * 16, 16)
            rot_re_v[dst] = rh * cr - ih * sr
            rot_im_v[dst] = rh * sr + ih * cr
        return carry

    lax.fori_loop(0, _B_PER_TILE, rot_body, 0)

    def start_chunk(c, buf, sem):
        pltpu.make_async_copy(
            ent_h.at[idx_v.at[pl.ds(pl.multiple_of(c * _CHUNK, _CHUNK),
                                    _CHUNK)]],
            buf, sem).start()

    def wait_chunk(buf, sem):
        pltpu.make_async_copy(
            ent_h.at[idx_v.at[pl.ds(0, _CHUNK)]], buf, sem).wait()

    def compute_chunk(buf, b, c):
        rotr = [rot_re_v[pl.ds(b * _HIDDEN + hv * 16, 16)]
                for hv in range(_HIDDEN // 16)]
        roti = [rot_im_v[pl.ds(b * _HIDDEN + hv * 16, 16)]
                for hv in range(_HIDDEN // 16)]

        def row_body(j, carry):
            sq = []
            for hv in range(_HIDDEN // 16):
                rt = buf[j, pl.ds(hv * 16, 16)]
                it = buf[j, pl.ds(_HIDDEN + hv * 16, 16)]
                d1 = rotr[hv] - rt
                d2 = roti[hv] - it
                s = d1 * d1 + (d2 * d2 + 1e-35)
                sq.append(_sqrt_fast(s))
            part = (((sq[0] + sq[1]) + (sq[2] + sq[3]))
                    + ((sq[4] + sq[5]) + (sq[6] + sq[7])))
            part_v[pl.ds(j * 16, 16)] = part
            return carry

        lax.fori_loop(0, _CHUNK, row_body, 0)
        pltpu.sync_copy(
            part_v,
            out_h.at[pl.ds((tb * _NNEG + c * _CHUNK) * 16, _CHUNK * 16)])

    start_chunk(0, buf0, sem_a)

    def pair_body(i, carry):
        start_chunk(2 * i + 1, buf1, sem_b)
        wait_chunk(buf0, sem_a)
        compute_chunk(buf0, i, 2 * i)

        @pl.when(i < _PAIRS - 1)
        def _():
            start_chunk(2 * i + 2, buf0, sem_a)

        wait_chunk(buf1, sem_b)
        compute_chunk(buf1, i, 2 * i + 1)
        return carry

    lax.fori_loop(0, _PAIRS, pair_body, 0)


@functools.lru_cache(maxsize=1)
def _build_scoring():
    return functools.partial(
        pl.kernel,
        out_type=jax.ShapeDtypeStruct((_NROWS * 16,), jnp.float32),
        scratch_types=[
            pltpu.VMEM((_B_PER_TILE,), jnp.int32),
            pltpu.VMEM((_B_PER_TILE,), jnp.int32),
            pltpu.VMEM((_ROWS_PER_TILE,), jnp.int32),
            pltpu.VMEM((_B_PER_TILE, _ENT_DIM), jnp.float32),
            pltpu.VMEM((_B_PER_TILE, _HIDDEN), jnp.float32),
            pltpu.VMEM((_B_PER_TILE * _HIDDEN,), jnp.float32),
            pltpu.VMEM((_B_PER_TILE * _HIDDEN,), jnp.float32),
            pltpu.VMEM((_CHUNK, _ENT_DIM), jnp.float32),
            pltpu.VMEM((_CHUNK, _ENT_DIM), jnp.float32),
            pltpu.VMEM((_CHUNK * 16,), jnp.float32),
            pltpu.SemaphoreType.DMA,
            pltpu.SemaphoreType.DMA,
            pltpu.SemaphoreType.DMA,
        ],
        mesh=plsc.VectorSubcoreMesh(core_axis_name="c", subcore_axis_name="s"),
    )(_sc_body)


_RED_BLK = 64


def _tc_reduce_body(part_ref, out_ref):
    x = part_ref[...]
    s = jnp.sum(x.reshape(_RED_BLK, 512, 16), axis=2)
    out_ref[...] = jnp.float32(_GAMMA) - s


@functools.lru_cache(maxsize=1)
def _build_reduce():
    grid = (_NROWS * 16) // (_RED_BLK * 8192)
    return pl.pallas_call(
        _tc_reduce_body,
        grid=(grid,),
        in_specs=[pl.BlockSpec((_RED_BLK, 8192), lambda i: (i, 0))],
        out_specs=pl.BlockSpec((_RED_BLK, 512), lambda i: (i, 0)),
        out_shape=jax.ShapeDtypeStruct((grid * _RED_BLK, 512), jnp.float32),
    )


@jax.jit
def kernel(head_idx, rel_idx, neg_tail_idx, entity_embedding,
           relation_embedding):
    tail_flat = neg_tail_idx.reshape(-1)
    part = _build_scoring()(head_idx, rel_idx, tail_flat, entity_embedding,
                            relation_embedding)
    out = _build_reduce()(part.reshape(_NROWS * 16 // 8192, 8192))
    return out.reshape(_B, _NNEG)
